# baseline (speedup 1.0000x reference)
import functools

import jax
import jax.numpy as jnp
import numpy as np
from jax import lax
from jax.experimental import pallas as pl
from jax.experimental.pallas import tpu as pltpu
from jax.experimental.pallas import tpu_sc as plsc

_H = 16
_K = 16
_CH = 16
_NC = 2
_NS = 16
_NW = _NC * _NS
_GBLK = 1024


def _nt(a, b):
    return lax.dot_general(a, b, (((1,), (1,)), ((), ())),
                           preferred_element_type=jnp.float32)


def _kv_proj_kernel(kb_ref, Wk_ref, bk_ref, Wv_ref, bv_ref, k_ref, v_ref):
    kb = kb_ref[0]
    k_ref[0] = _nt(kb, Wk_ref[...]) + bk_ref[...]
    v_ref[0] = _nt(kb, Wv_ref[...]) + bv_ref[...]


def _scores_kernel(x_ref, k_ref, Wq_ref, bq_ref, s_ref, gi_ref,
                   *, inv_scale):
    q = _nt(x_ref[0], Wq_ref[...]) + bq_ref[...]
    kk = k_ref[0]
    sblk = q.shape[0]
    hd = q.shape[1] // _H
    n_chunks = kk.shape[0] // _CH
    iota_c = lax.broadcasted_iota(jnp.int32, (sblk, n_chunks), 1)
    iota_k = lax.broadcasted_iota(jnp.int32, (sblk, _K), 1)
    for h in range(_H):
        sl = slice(h * hd, (h + 1) * hd)
        s = _nt(q[:, sl], kk[:, sl]) * inv_scale
        s_ref[0, h] = s
        w = jnp.max(s.reshape(sblk, n_chunks, _CH), axis=2)
        li = jnp.zeros((sblk, _K), jnp.int32)
        for k in range(_K):
            t = jnp.max(w, axis=1, keepdims=True)
            idx = jnp.min(jnp.where(w >= t, iota_c, n_chunks),
                          axis=1, keepdims=True)
            li = jnp.where(iota_k == k, idx, li)
            w = jnp.where(iota_c == idx, -jnp.inf, w)
        gi_ref[0, h] = li


def _sc_gather_kernel(s_ref, li_ref, cand_ref, st_v, idx_v, out_v,
                      *, rows_per_w, rblk, n):
    wid = lax.axis_index("s") * _NC + lax.axis_index("c")
    base_r = wid * rows_per_w
    iota = lax.iota(jnp.int32, _CH)

    def blk_body(blk, _):
        r0 = pl.multiple_of(base_r + blk * rblk, rblk)
        pltpu.sync_copy(s_ref.at[pl.ds(r0 * n, rblk * n)], st_v)
        pltpu.sync_copy(li_ref.at[pl.ds(r0 * _K, rblk * _K)], idx_v)

        def row_body(r, _):
            irow = idx_v[pl.ds(r * _K, _K)]
            base = r * n
            for j in range(_K):
                c = irow[j]
                v = plsc.load_gather(st_v, [base + c * _CH + iota])
                out_v[pl.ds((r * _K + j) * _CH, _CH)] = v
            return 0

        lax.fori_loop(0, rblk, row_body, 0)
        pltpu.sync_copy(out_v, cand_ref.at[pl.ds(r0 * _K * _CH,
                                                 rblk * _K * _CH)])
        return 0

    lax.fori_loop(0, rows_per_w // rblk, blk_body, 0)


def _combine_kernel(s_ref, c_ref, v_ref, Wo_ref, bo_ref, out_ref, ctx_ref):
    vv = v_ref[0]
    hd = ctx_ref.shape[1] // _H
    for h in range(_H):
        s = s_ref[0, h]
        w = c_ref[0, h]
        t = None
        for _ in range(_K):
            t = jnp.max(w, axis=1, keepdims=True)
            w = jnp.where(w >= t, -jnp.inf, w)
        p = jnp.where(s >= t, jnp.exp(s - t), 0.0)
        denom = jnp.sum(p, axis=1, keepdims=True)
        sl = slice(h * hd, (h + 1) * hd)
        ctx_ref[:, sl] = jnp.dot(
            p, vv[:, sl], preferred_element_type=jnp.float32) / denom
    out_ref[0] = _nt(ctx_ref[...], Wo_ref[...]) + bo_ref[...]


@jax.jit
def kernel(x, knowledge_embeddings, Wq, bq, Wk, bk, Wv, bv, Wo, bo):
    B, S, D = x.shape
    N = knowledge_embeddings.shape[1]
    inv_scale = float(1.0 / np.sqrt(D // _H))
    R = B * _H * S
    NI = R * _K

    full2 = pl.BlockSpec((D, D), lambda *_: (0, 0))
    full1 = pl.BlockSpec((1, D), lambda *_: (0, 0))
    bnd = pl.BlockSpec((1, N, D), lambda b, *_: (b, 0, 0))

    k_proj, v_proj = pl.pallas_call(
        _kv_proj_kernel,
        grid=(B,),
        in_specs=[bnd, full2, full1, full2, full1],
        out_specs=[bnd, bnd],
        out_shape=[jax.ShapeDtypeStruct((B, N, D), jnp.float32)] * 2,
    )(knowledge_embeddings, Wk, bk.reshape(1, D), Wv, bv.reshape(1, D))

    sblk_b = 128
    scores, gidx = pl.pallas_call(
        functools.partial(_scores_kernel, inv_scale=inv_scale),
        grid=(B, S // sblk_b),
        in_specs=[
            pl.BlockSpec((1, sblk_b, D), lambda b, s: (b, s, 0)),
            bnd, full2, full1,
        ],
        out_specs=[
            pl.BlockSpec((1, _H, sblk_b, N), lambda b, s: (b, 0, s, 0)),
            pl.BlockSpec((1, _H, sblk_b, _K), lambda b, s: (b, 0, s, 0)),
        ],
        out_shape=[
            jax.ShapeDtypeStruct((B, _H, S, N), jnp.float32),
            jax.ShapeDtypeStruct((B, _H, S, _K), jnp.int32),
        ],
    )(x, k_proj, Wq, bq.reshape(1, D))

    rblk = 64
    mesh = plsc.VectorSubcoreMesh(core_axis_name="c", subcore_axis_name="s")
    cand = pl.kernel(
        functools.partial(_sc_gather_kernel, rows_per_w=R // _NW, rblk=rblk,
                          n=N),
        mesh=mesh,
        compiler_params=pltpu.CompilerParams(needs_layout_passes=False),
        out_type=jax.ShapeDtypeStruct((R * _K * _CH,), jnp.float32),
        scratch_types=[
            pltpu.VMEM((rblk * N,), jnp.float32),
            pltpu.VMEM((rblk * _K,), jnp.int32),
            pltpu.VMEM((rblk * _K * _CH,), jnp.float32),
        ],
    )(scores.reshape(R * N), gidx.reshape(R * _K))

    sblk_d = 128
    out = pl.pallas_call(
        _combine_kernel,
        grid=(B, S // sblk_d),
        in_specs=[
            pl.BlockSpec((1, _H, sblk_d, N), lambda b, s: (b, 0, s, 0)),
            pl.BlockSpec((1, _H, sblk_d, _K * _CH), lambda b, s: (b, 0, s, 0)),
            bnd, full2, full1,
        ],
        out_specs=pl.BlockSpec((1, sblk_d, D), lambda b, s: (b, s, 0)),
        out_shape=jax.ShapeDtypeStruct((B, S, D), jnp.float32),
        scratch_shapes=[pltpu.VMEM((sblk_d, D), jnp.float32)],
    )(scores, cand.reshape(B, _H, S, _K * _CH), v_proj, Wo, bo.reshape(1, D))
    return out


# speedup vs baseline: 1.5316x; 1.5316x over previous
import functools

import jax
import jax.numpy as jnp
import numpy as np
from jax import lax
from jax.experimental import pallas as pl
from jax.experimental.pallas import tpu as pltpu
from jax.experimental.pallas import tpu_sc as plsc

_H = 16
_K = 16
_VW = 128
_NC = 2
_NS = 16
_NW = _NC * _NS


def _nt(a, b):
    return lax.dot_general(a, b, (((1,), (1,)), ((), ())),
                           preferred_element_type=jnp.float32)


def _kv_proj_kernel(kb_ref, Wk_ref, bk_ref, Wv_ref, bv_ref, k_ref, v_ref):
    kb = kb_ref[0]
    k_ref[0] = _nt(kb, Wk_ref[...]) + bk_ref[...]
    v_ref[0] = _nt(kb, Wv_ref[...]) + bv_ref[...]


def _topk_kernel(x_ref, k_ref, Wq_ref, bq_ref, wgt_ref, vid_ref,
                 *, inv_scale, n_total):
    q = _nt(x_ref[0], Wq_ref[...]) + bq_ref[...]
    kk = k_ref[0]
    b = pl.program_id(0)
    sblk = q.shape[0]
    hd = q.shape[1] // _H
    n = kk.shape[0]
    iota_n = lax.broadcasted_iota(jnp.int32, (sblk, n), 1)
    iota_k = lax.broadcasted_iota(jnp.int32, (sblk, _K), 1)
    for h in range(_H):
        sl = slice(h * hd, (h + 1) * hd)
        w = _nt(q[:, sl], kk[:, sl]) * inv_scale
        li = jnp.zeros((sblk, _K), jnp.int32)
        lv = jnp.zeros((sblk, _K), jnp.float32)
        for k in range(_K):
            t = jnp.max(w, axis=1, keepdims=True)
            idx = jnp.min(jnp.where(w >= t, iota_n, n),
                          axis=1, keepdims=True)
            li = jnp.where(iota_k == k, idx, li)
            lv = jnp.where(iota_k == k, t, lv)
            w = jnp.where(iota_n == idx, -jnp.inf, w)
        m = jnp.max(lv, axis=1, keepdims=True)
        e = jnp.exp(lv - m)
        wgt_ref[0, h] = e / jnp.sum(e, axis=1, keepdims=True)
        vid_ref[0, h] = (b * n_total + li) * (_H // 2) + (h // 2)


def _sc_gather_kernel(tbl_ref, gi_ref, cand_ref, idx_v, rows_v, sem,
                      *, ni_per_w, iblk):
    wid = lax.axis_index("s") * _NC + lax.axis_index("c")
    base = wid * ni_per_w

    def blk_body(blk, _):
        o = pl.multiple_of(base + blk * iblk, iblk)
        pltpu.sync_copy(gi_ref.at[pl.ds(o, iblk)], idx_v)
        pltpu.async_copy(tbl_ref.at[idx_v], rows_v, sem).wait()
        pltpu.sync_copy(rows_v, cand_ref.at[pl.ds(o, iblk)])
        return 0

    lax.fori_loop(0, ni_per_w // iblk, blk_body, 0)


def _combine_kernel(w_ref, c_ref, Wo_ref, bo_ref, out_ref):
    hp = pl.program_id(2)
    sblk = out_ref.shape[1]
    parts = []
    for j in range(2):
        w = w_ref[0, j]
        c = c_ref[0, j]
        acc = jnp.zeros((sblk, 64), jnp.float32)
        for k in range(_K):
            off = k * _VW + j * 64
            acc = acc + w[:, k:k + 1] * c[:, off:off + 64]
        parts.append(acc)
    ctx = jnp.concatenate(parts, axis=1)
    contrib = _nt(ctx, Wo_ref[...])

    @pl.when(hp == 0)
    def _():
        out_ref[0] = contrib + bo_ref[...]

    @pl.when(hp != 0)
    def _():
        out_ref[0] = out_ref[0] + contrib


@jax.jit
def kernel(x, knowledge_embeddings, Wq, bq, Wk, bk, Wv, bv, Wo, bo):
    B, S, D = x.shape
    N = knowledge_embeddings.shape[1]
    inv_scale = float(1.0 / np.sqrt(D // _H))
    R = B * _H * S
    NI = R * _K

    full2 = pl.BlockSpec((D, D), lambda *_: (0, 0))
    full1 = pl.BlockSpec((1, D), lambda *_: (0, 0))
    bnd = pl.BlockSpec((1, N, D), lambda b, *_: (b, 0, 0))

    k_proj, v_proj = pl.pallas_call(
        _kv_proj_kernel,
        grid=(B,),
        in_specs=[bnd, full2, full1, full2, full1],
        out_specs=[bnd, bnd],
        out_shape=[jax.ShapeDtypeStruct((B, N, D), jnp.float32)] * 2,
    )(knowledge_embeddings, Wk, bk.reshape(1, D), Wv, bv.reshape(1, D))

    sblk_b = 128
    wgt, vid = pl.pallas_call(
        functools.partial(_topk_kernel, inv_scale=inv_scale, n_total=N),
        grid=(B, S // sblk_b),
        in_specs=[
            pl.BlockSpec((1, sblk_b, D), lambda b, s: (b, s, 0)),
            bnd, full2, full1,
        ],
        out_specs=[
            pl.BlockSpec((1, _H, sblk_b, _K), lambda b, s: (b, 0, s, 0)),
            pl.BlockSpec((1, _H, sblk_b, _K), lambda b, s: (b, 0, s, 0)),
        ],
        out_shape=[
            jax.ShapeDtypeStruct((B, _H, S, _K), jnp.float32),
            jax.ShapeDtypeStruct((B, _H, S, _K), jnp.int32),
        ],
    )(x, k_proj, Wq, bq.reshape(1, D))

    iblk = 512
    mesh = plsc.VectorSubcoreMesh(core_axis_name="c", subcore_axis_name="s")
    cand = pl.kernel(
        functools.partial(_sc_gather_kernel, ni_per_w=NI // _NW, iblk=iblk),
        mesh=mesh,
        out_type=jax.ShapeDtypeStruct((NI, _VW), jnp.float32),
        scratch_types=[
            pltpu.VMEM((iblk,), jnp.int32),
            pltpu.VMEM((iblk, _VW), jnp.float32),
            pltpu.SemaphoreType.DMA,
        ],
    )(v_proj.reshape(B * N * (_H // 2), _VW), vid.reshape(NI))

    sblk_d = 128
    out = pl.pallas_call(
        _combine_kernel,
        grid=(B, S // sblk_d, _H // 2),
        in_specs=[
            pl.BlockSpec((1, 2, sblk_d, _K), lambda b, s, hp: (b, hp, s, 0)),
            pl.BlockSpec((1, 2, sblk_d, _K * _VW),
                         lambda b, s, hp: (b, hp, s, 0)),
            pl.BlockSpec((D, _VW), lambda b, s, hp: (0, hp)),
            pl.BlockSpec((1, D), lambda *_: (0, 0)),
        ],
        out_specs=pl.BlockSpec((1, sblk_d, D), lambda b, s, hp: (b, s, 0)),
        out_shape=jax.ShapeDtypeStruct((B, S, D), jnp.float32),
    )(wgt, cand.reshape(B, _H, S, _K * _VW), Wo, bo.reshape(1, D))
    return out


# speedup vs baseline: 1.5430x; 1.0075x over previous
import functools

import jax
import jax.numpy as jnp
import numpy as np
from jax import lax
from jax.experimental import pallas as pl
from jax.experimental.pallas import tpu as pltpu
from jax.experimental.pallas import tpu_sc as plsc

_H = 16
_K = 16
_VW = 128
_NC = 2
_NS = 16
_NW = _NC * _NS


def _nt(a, b):
    return lax.dot_general(a, b, (((1,), (1,)), ((), ())),
                           preferred_element_type=jnp.float32)


def _kv_proj_kernel(kb_ref, Wk_ref, bk_ref, Wv_ref, bv_ref, k_ref, v_ref):
    kb = kb_ref[0]
    k_ref[0] = _nt(kb, Wk_ref[...]) + bk_ref[...]
    v_ref[0] = _nt(kb, Wv_ref[...]) + bv_ref[...]


def _topk_kernel(x_ref, k_ref, Wq_ref, bq_ref, wgt_ref, vid_ref,
                 *, inv_scale, n_total):
    q = _nt(x_ref[0], Wq_ref[...]) + bq_ref[...]
    kk = k_ref[0]
    b = pl.program_id(0)
    sblk = q.shape[0]
    hd = q.shape[1] // _H
    n = kk.shape[0]
    iota_n = lax.broadcasted_iota(jnp.int32, (sblk, n), 1)
    iota_k = lax.broadcasted_iota(jnp.int32, (sblk, _K), 1)
    for h in range(_H):
        sl = slice(h * hd, (h + 1) * hd)
        w = _nt(q[:, sl], kk[:, sl]) * inv_scale
        li = jnp.zeros((sblk, _K), jnp.int32)
        lv = jnp.zeros((sblk, _K), jnp.float32)
        for k in range(_K):
            t = jnp.max(w, axis=1, keepdims=True)
            idx = jnp.min(jnp.where(w >= t, iota_n, n),
                          axis=1, keepdims=True)
            li = jnp.where(iota_k == k, idx, li)
            lv = jnp.where(iota_k == k, t, lv)
            w = jnp.where(iota_n == idx, -jnp.inf, w)
        m = jnp.max(lv, axis=1, keepdims=True)
        e = jnp.exp(lv - m)
        wgt_ref[0, h] = e / jnp.sum(e, axis=1, keepdims=True)
        vid_ref[0, h] = (b * n_total + li) * (_H // 2) + (h // 2)


def _sc_gather_kernel(tbl_ref, gi_ref, cand_ref, idx0, idx1, rows0, rows1,
                      sem0, sem1, *, ni_per_w, iblk):
    wid = lax.axis_index("s") * _NC + lax.axis_index("c")
    base = wid * ni_per_w

    def blk_body(blk, _):
        o0 = pl.multiple_of(base + 2 * blk * iblk, iblk)
        o1 = pl.multiple_of(base + (2 * blk + 1) * iblk, iblk)
        pltpu.sync_copy(gi_ref.at[pl.ds(o0, iblk)], idx0)
        cp0 = pltpu.async_copy(tbl_ref.at[idx0], rows0, sem0)
        pltpu.sync_copy(gi_ref.at[pl.ds(o1, iblk)], idx1)
        cp1 = pltpu.async_copy(tbl_ref.at[idx1], rows1, sem1)
        cp0.wait()
        pltpu.sync_copy(rows0, cand_ref.at[pl.ds(o0, iblk)])
        cp1.wait()
        pltpu.sync_copy(rows1, cand_ref.at[pl.ds(o1, iblk)])
        return 0

    lax.fori_loop(0, ni_per_w // (2 * iblk), blk_body, 0)


def _combine_kernel(w_ref, c_ref, Wo_ref, bo_ref, out_ref):
    hp = pl.program_id(2)
    sblk = out_ref.shape[1]
    parts = []
    for j in range(2):
        w = w_ref[0, j]
        c = c_ref[0, j]
        acc = jnp.zeros((sblk, 64), jnp.float32)
        for k in range(_K):
            off = k * _VW + j * 64
            acc = acc + w[:, k:k + 1] * c[:, off:off + 64]
        parts.append(acc)
    ctx = jnp.concatenate(parts, axis=1)
    contrib = _nt(ctx, Wo_ref[...])

    @pl.when(hp == 0)
    def _():
        out_ref[0] = contrib + bo_ref[...]

    @pl.when(hp != 0)
    def _():
        out_ref[0] = out_ref[0] + contrib


@jax.jit
def kernel(x, knowledge_embeddings, Wq, bq, Wk, bk, Wv, bv, Wo, bo):
    B, S, D = x.shape
    N = knowledge_embeddings.shape[1]
    inv_scale = float(1.0 / np.sqrt(D // _H))
    R = B * _H * S
    NI = R * _K

    full2 = pl.BlockSpec((D, D), lambda *_: (0, 0))
    full1 = pl.BlockSpec((1, D), lambda *_: (0, 0))
    bnd = pl.BlockSpec((1, N, D), lambda b, *_: (b, 0, 0))

    k_proj, v_proj = pl.pallas_call(
        _kv_proj_kernel,
        grid=(B,),
        in_specs=[bnd, full2, full1, full2, full1],
        out_specs=[bnd, bnd],
        out_shape=[jax.ShapeDtypeStruct((B, N, D), jnp.float32)] * 2,
    )(knowledge_embeddings, Wk, bk.reshape(1, D), Wv, bv.reshape(1, D))

    sblk_b = 128
    wgt, vid = pl.pallas_call(
        functools.partial(_topk_kernel, inv_scale=inv_scale, n_total=N),
        grid=(B, S // sblk_b),
        in_specs=[
            pl.BlockSpec((1, sblk_b, D), lambda b, s: (b, s, 0)),
            bnd, full2, full1,
        ],
        out_specs=[
            pl.BlockSpec((1, _H, sblk_b, _K), lambda b, s: (b, 0, s, 0)),
            pl.BlockSpec((1, _H, sblk_b, _K), lambda b, s: (b, 0, s, 0)),
        ],
        out_shape=[
            jax.ShapeDtypeStruct((B, _H, S, _K), jnp.float32),
            jax.ShapeDtypeStruct((B, _H, S, _K), jnp.int32),
        ],
    )(x, k_proj, Wq, bq.reshape(1, D))

    iblk = 256
    mesh = plsc.VectorSubcoreMesh(core_axis_name="c", subcore_axis_name="s")
    cand = pl.kernel(
        functools.partial(_sc_gather_kernel, ni_per_w=NI // _NW, iblk=iblk),
        mesh=mesh,
        out_type=jax.ShapeDtypeStruct((NI, _VW), jnp.float32),
        scratch_types=[
            pltpu.VMEM((iblk,), jnp.int32),
            pltpu.VMEM((iblk,), jnp.int32),
            pltpu.VMEM((iblk, _VW), jnp.float32),
            pltpu.VMEM((iblk, _VW), jnp.float32),
            pltpu.SemaphoreType.DMA,
            pltpu.SemaphoreType.DMA,
        ],
    )(v_proj.reshape(B * N * (_H // 2), _VW), vid.reshape(NI))

    sblk_d = 128
    out = pl.pallas_call(
        _combine_kernel,
        grid=(B, S // sblk_d, _H // 2),
        in_specs=[
            pl.BlockSpec((1, 2, sblk_d, _K), lambda b, s, hp: (b, hp, s, 0)),
            pl.BlockSpec((1, 2, sblk_d, _K * _VW),
                         lambda b, s, hp: (b, hp, s, 0)),
            pl.BlockSpec((D, _VW), lambda b, s, hp: (0, hp)),
            pl.BlockSpec((1, D), lambda *_: (0, 0)),
        ],
        out_specs=pl.BlockSpec((1, sblk_d, D), lambda b, s, hp: (b, s, 0)),
        out_shape=jax.ShapeDtypeStruct((B, S, D), jnp.float32),
    )(wgt, cand.reshape(B, _H, S, _K * _VW), Wo, bo.reshape(1, D))
    return out


# speedup vs baseline: 1.9616x; 1.2713x over previous
import functools

import jax
import jax.numpy as jnp
import numpy as np
from jax import lax
from jax.experimental import pallas as pl
from jax.experimental.pallas import tpu as pltpu
from jax.experimental.pallas import tpu_sc as plsc

_H = 16
_K = 16
_VW = 128
_NC = 2
_NS = 16
_NW = _NC * _NS


def _nt(a, b):
    return lax.dot_general(a, b, (((1,), (1,)), ((), ())),
                           preferred_element_type=jnp.float32)


def _kv_proj_kernel(kb_ref, Wk_ref, bk_ref, Wv_ref, bv_ref, k_ref, v_ref):
    kb = kb_ref[0]
    k_ref[0] = _nt(kb, Wk_ref[...]) + bk_ref[...]
    v_ref[0] = _nt(kb, Wv_ref[...]) + bv_ref[...]


def _topk_kernel(x_ref, k_ref, Wq_ref, bq_ref, wgt_ref, vid_ref,
                 *, inv_scale, n_total):
    q = _nt(x_ref[0], Wq_ref[...]) + bq_ref[...]
    kk = k_ref[0]
    b = pl.program_id(0)
    sblk = q.shape[0]
    hd = q.shape[1] // _H
    n = kk.shape[0]
    iota_n = lax.broadcasted_iota(jnp.int32, (sblk, n), 1)
    iota_k = lax.broadcasted_iota(jnp.int32, (sblk, _K), 1)
    for h in range(_H):
        sl = slice(h * hd, (h + 1) * hd)
        w = _nt(q[:, sl], kk[:, sl]) * inv_scale
        li = jnp.zeros((sblk, _K), jnp.int32)
        lv = jnp.zeros((sblk, _K), jnp.float32)
        for k in range(_K):
            t = jnp.max(w, axis=1, keepdims=True)
            idx = jnp.min(jnp.where(w >= t, iota_n, n),
                          axis=1, keepdims=True)
            li = jnp.where(iota_k == k, idx, li)
            lv = jnp.where(iota_k == k, t, lv)
            w = jnp.where(iota_n == idx, -jnp.inf, w)
        m = jnp.max(lv, axis=1, keepdims=True)
        e = jnp.exp(lv - m)
        wgt_ref[0, h] = e / jnp.sum(e, axis=1, keepdims=True)
        vid_ref[0, h] = (b * n_total + li) * (_H // 2) + (h // 2)


def _sc_gather_kernel(tbl_ref, gi_ref, cand_ref, idx0, idx1, rows0, rows1,
                      sem0, sem1, *, ni_per_w, iblk):
    wid = lax.axis_index("s") * _NC + lax.axis_index("c")
    base = wid * ni_per_w

    def blk_body(blk, _):
        o0 = pl.multiple_of(base + 2 * blk * iblk, iblk)
        o1 = pl.multiple_of(base + (2 * blk + 1) * iblk, iblk)
        pltpu.sync_copy(gi_ref.at[pl.ds(o0, iblk)], idx0)
        cp0 = pltpu.async_copy(tbl_ref.at[idx0], rows0, sem0)
        pltpu.sync_copy(gi_ref.at[pl.ds(o1, iblk)], idx1)
        cp1 = pltpu.async_copy(tbl_ref.at[idx1], rows1, sem1)
        cp0.wait()
        pltpu.sync_copy(rows0, cand_ref.at[pl.ds(o0, iblk)])
        cp1.wait()
        pltpu.sync_copy(rows1, cand_ref.at[pl.ds(o1, iblk)])
        return 0

    lax.fori_loop(0, ni_per_w // (2 * iblk), blk_body, 0)


def _combine_kernel(w_ref, c_ref, Wo_ref, bo_ref, out_ref):
    hp = pl.program_id(2)
    sblk = out_ref.shape[1]
    parts = []
    for j in range(2):
        w = w_ref[0, j]
        c = c_ref[0, j]
        acc = jnp.zeros((sblk, 64), jnp.float32)
        for k in range(_K):
            off = k * _VW + j * 64
            acc = acc + w[:, k:k + 1] * c[:, off:off + 64]
        parts.append(acc)
    ctx = jnp.concatenate(parts, axis=1)
    contrib = _nt(ctx, Wo_ref[...])

    @pl.when(hp == 0)
    def _():
        out_ref[0] = contrib + bo_ref[...]

    @pl.when(hp != 0)
    def _():
        out_ref[0] = out_ref[0] + contrib


@jax.jit
def kernel(x, knowledge_embeddings, Wq, bq, Wk, bk, Wv, bv, Wo, bo):
    B, S, D = x.shape
    N = knowledge_embeddings.shape[1]
    inv_scale = float(1.0 / np.sqrt(D // _H))
    R = B * _H * S
    NI = R * _K

    full2 = pl.BlockSpec((D, D), lambda *_: (0, 0))
    full1 = pl.BlockSpec((1, D), lambda *_: (0, 0))
    bnd = pl.BlockSpec((1, N, D), lambda b, *_: (b, 0, 0))

    k_proj, v_proj = pl.pallas_call(
        _kv_proj_kernel,
        grid=(B,),
        in_specs=[bnd, full2, full1, full2, full1],
        out_specs=[bnd, bnd],
        out_shape=[jax.ShapeDtypeStruct((B, N, D), jnp.float32)] * 2,
    )(knowledge_embeddings, Wk, bk.reshape(1, D), Wv, bv.reshape(1, D))

    sblk_b = 512
    wgt, vid = pl.pallas_call(
        functools.partial(_topk_kernel, inv_scale=inv_scale, n_total=N),
        grid=(B, S // sblk_b),
        in_specs=[
            pl.BlockSpec((1, sblk_b, D), lambda b, s: (b, s, 0)),
            bnd, full2, full1,
        ],
        out_specs=[
            pl.BlockSpec((1, _H, sblk_b, _K), lambda b, s: (b, 0, s, 0)),
            pl.BlockSpec((1, _H, sblk_b, _K), lambda b, s: (b, 0, s, 0)),
        ],
        out_shape=[
            jax.ShapeDtypeStruct((B, _H, S, _K), jnp.float32),
            jax.ShapeDtypeStruct((B, _H, S, _K), jnp.int32),
        ],
    )(x, k_proj, Wq, bq.reshape(1, D))

    iblk = 256
    mesh = plsc.VectorSubcoreMesh(core_axis_name="c", subcore_axis_name="s")
    cand = pl.kernel(
        functools.partial(_sc_gather_kernel, ni_per_w=NI // _NW, iblk=iblk),
        mesh=mesh,
        out_type=jax.ShapeDtypeStruct((NI, _VW), jnp.float32),
        scratch_types=[
            pltpu.VMEM((iblk,), jnp.int32),
            pltpu.VMEM((iblk,), jnp.int32),
            pltpu.VMEM((iblk, _VW), jnp.float32),
            pltpu.VMEM((iblk, _VW), jnp.float32),
            pltpu.SemaphoreType.DMA,
            pltpu.SemaphoreType.DMA,
        ],
    )(v_proj.reshape(B * N * (_H // 2), _VW), vid.reshape(NI))

    sblk_d = 128
    out = pl.pallas_call(
        _combine_kernel,
        grid=(B, S // sblk_d, _H // 2),
        in_specs=[
            pl.BlockSpec((1, 2, sblk_d, _K), lambda b, s, hp: (b, hp, s, 0)),
            pl.BlockSpec((1, 2, sblk_d, _K * _VW),
                         lambda b, s, hp: (b, hp, s, 0)),
            pl.BlockSpec((D, _VW), lambda b, s, hp: (0, hp)),
            pl.BlockSpec((1, D), lambda *_: (0, 0)),
        ],
        out_specs=pl.BlockSpec((1, sblk_d, D), lambda b, s, hp: (b, s, 0)),
        out_shape=jax.ShapeDtypeStruct((B, S, D), jnp.float32),
    )(wgt, cand.reshape(B, _H, S, _K * _VW), Wo, bo.reshape(1, D))
    return out


# speedup vs baseline: 2.0603x; 1.0503x over previous
import functools

import jax
import jax.numpy as jnp
import numpy as np
from jax import lax
from jax.experimental import pallas as pl
from jax.experimental.pallas import tpu as pltpu
from jax.experimental.pallas import tpu_sc as plsc

_H = 16
_K = 16
_VW = 128
_NC = 2
_NS = 16
_NW = _NC * _NS


def _nt(a, b):
    return lax.dot_general(a, b, (((1,), (1,)), ((), ())),
                           preferred_element_type=jnp.float32)


def _kv_proj_kernel(kb_ref, Wk_ref, bk_ref, Wv_ref, bv_ref, k_ref, v_ref):
    kb = kb_ref[0]
    k_ref[0] = _nt(kb, Wk_ref[...]) + bk_ref[...]
    v_ref[0] = _nt(kb, Wv_ref[...]) + bv_ref[...]


def _topk_kernel(x_ref, k_ref, Wq_ref, bq_ref, wgt_ref, vid_ref,
                 *, inv_scale, n_total):
    q = _nt(x_ref[0], Wq_ref[...]) + bq_ref[...]
    kk = k_ref[0]
    b = pl.program_id(0)
    sblk = q.shape[0]
    hd = q.shape[1] // _H
    n = kk.shape[0]
    iota_n = lax.broadcasted_iota(jnp.int32, (sblk, n), 1)
    iota_k = lax.broadcasted_iota(jnp.int32, (sblk, _K), 1)
    for h in range(_H):
        sl = slice(h * hd, (h + 1) * hd)
        w = _nt(q[:, sl], kk[:, sl]) * inv_scale
        li = jnp.zeros((sblk, _K), jnp.int32)
        lv = jnp.zeros((sblk, _K), jnp.float32)
        for k in range(_K):
            t = jnp.max(w, axis=1, keepdims=True)
            idx = jnp.min(jnp.where(w >= t, iota_n, n),
                          axis=1, keepdims=True)
            li = jnp.where(iota_k == k, idx, li)
            lv = jnp.where(iota_k == k, t, lv)
            w = jnp.where(iota_n == idx, -jnp.inf, w)
        m = jnp.max(lv, axis=1, keepdims=True)
        e = jnp.exp(lv - m)
        wgt_ref[0, h] = e / jnp.sum(e, axis=1, keepdims=True)
        vid_ref[0, h] = (b * n_total + li) * (_H // 2) + (h // 2)


def _sc_gather_kernel(tbl_ref, gi_ref, cand_ref, idx0, idx1, rows0, rows1,
                      sem0, sem1, *, ni_per_w, iblk):
    wid = lax.axis_index("s") * _NC + lax.axis_index("c")
    base = wid * ni_per_w

    def blk_body(blk, _):
        o0 = pl.multiple_of(base + 2 * blk * iblk, iblk)
        o1 = pl.multiple_of(base + (2 * blk + 1) * iblk, iblk)
        pltpu.sync_copy(gi_ref.at[pl.ds(o0, iblk)], idx0)
        cp0 = pltpu.async_copy(tbl_ref.at[idx0], rows0, sem0)
        pltpu.sync_copy(gi_ref.at[pl.ds(o1, iblk)], idx1)
        cp1 = pltpu.async_copy(tbl_ref.at[idx1], rows1, sem1)
        cp0.wait()
        pltpu.sync_copy(rows0, cand_ref.at[pl.ds(o0, iblk)])
        cp1.wait()
        pltpu.sync_copy(rows1, cand_ref.at[pl.ds(o1, iblk)])
        return 0

    lax.fori_loop(0, ni_per_w // (2 * iblk), blk_body, 0)


def _combine_kernel(w_ref, c_ref, Wo_ref, bo_ref, out_ref):
    hp = pl.program_id(2)
    sblk = out_ref.shape[1]
    parts = []
    for j in range(2):
        w = w_ref[0, j]
        c = c_ref[0, j]
        acc = jnp.zeros((sblk, 64), jnp.float32)
        for k in range(_K):
            off = k * _VW + j * 64
            acc = acc + w[:, k:k + 1] * c[:, off:off + 64]
        parts.append(acc)
    ctx = jnp.concatenate(parts, axis=1)
    contrib = _nt(ctx, Wo_ref[...])

    @pl.when(hp == 0)
    def _():
        out_ref[0] = contrib + bo_ref[...]

    @pl.when(hp != 0)
    def _():
        out_ref[0] = out_ref[0] + contrib


@jax.jit
def kernel(x, knowledge_embeddings, Wq, bq, Wk, bk, Wv, bv, Wo, bo):
    B, S, D = x.shape
    N = knowledge_embeddings.shape[1]
    inv_scale = float(1.0 / np.sqrt(D // _H))
    R = B * _H * S
    NI = R * _K

    full2 = pl.BlockSpec((D, D), lambda *_: (0, 0))
    full1 = pl.BlockSpec((1, D), lambda *_: (0, 0))
    bnd = pl.BlockSpec((1, N, D), lambda b, *_: (b, 0, 0))

    k_proj, v_proj = pl.pallas_call(
        _kv_proj_kernel,
        grid=(B,),
        in_specs=[bnd, full2, full1, full2, full1],
        out_specs=[bnd, bnd],
        out_shape=[jax.ShapeDtypeStruct((B, N, D), jnp.float32)] * 2,
    )(knowledge_embeddings, Wk, bk.reshape(1, D), Wv, bv.reshape(1, D))

    sblk_b = 512
    wgt, vid = pl.pallas_call(
        functools.partial(_topk_kernel, inv_scale=inv_scale, n_total=N),
        grid=(B, S // sblk_b),
        in_specs=[
            pl.BlockSpec((1, sblk_b, D), lambda b, s: (b, s, 0)),
            bnd, full2, full1,
        ],
        out_specs=[
            pl.BlockSpec((1, _H, sblk_b, _K), lambda b, s: (b, 0, s, 0)),
            pl.BlockSpec((1, _H, sblk_b, _K), lambda b, s: (b, 0, s, 0)),
        ],
        out_shape=[
            jax.ShapeDtypeStruct((B, _H, S, _K), jnp.float32),
            jax.ShapeDtypeStruct((B, _H, S, _K), jnp.int32),
        ],
    )(x, k_proj, Wq, bq.reshape(1, D))

    iblk = 256
    mesh = plsc.VectorSubcoreMesh(core_axis_name="c", subcore_axis_name="s")
    cand = pl.kernel(
        functools.partial(_sc_gather_kernel, ni_per_w=NI // _NW, iblk=iblk),
        mesh=mesh,
        out_type=jax.ShapeDtypeStruct((NI, _VW), jnp.float32),
        scratch_types=[
            pltpu.VMEM((iblk,), jnp.int32),
            pltpu.VMEM((iblk,), jnp.int32),
            pltpu.VMEM((iblk, _VW), jnp.float32),
            pltpu.VMEM((iblk, _VW), jnp.float32),
            pltpu.SemaphoreType.DMA,
            pltpu.SemaphoreType.DMA,
        ],
    )(v_proj.reshape(B * N * (_H // 2), _VW), vid.reshape(NI))

    sblk_d = 512
    out = pl.pallas_call(
        _combine_kernel,
        grid=(B, S // sblk_d, _H // 2),
        in_specs=[
            pl.BlockSpec((1, 2, sblk_d, _K), lambda b, s, hp: (b, hp, s, 0)),
            pl.BlockSpec((1, 2, sblk_d, _K * _VW),
                         lambda b, s, hp: (b, hp, s, 0)),
            pl.BlockSpec((D, _VW), lambda b, s, hp: (0, hp)),
            pl.BlockSpec((1, D), lambda *_: (0, 0)),
        ],
        out_specs=pl.BlockSpec((1, sblk_d, D), lambda b, s, hp: (b, s, 0)),
        out_shape=jax.ShapeDtypeStruct((B, S, D), jnp.float32),
    )(wgt, cand.reshape(B, _H, S, _K * _VW), Wo, bo.reshape(1, D))
    return out
